# baseline (speedup 1.0000x reference)
import jax
import jax.numpy as jnp
from jax import lax
from jax.experimental import pallas as pl
from jax.experimental.pallas import tpu as pltpu
from jax.experimental.pallas import tpu_sc as plsc

B, L, D = 4096, 200, 128
N = B * L
NC, NS = 2, 16
NW = NC * NS
PER_W = N // NW
AV, NV = 121, 17
CH = 128
NCHUNK = PER_W // CH
NBUF = 2
DQ = 8


def _gather_body(atoms_hbm, neigh_hbm, at_hbm, nt_hbm, out_hbm,
                 at_v, nt_v, ia_v, in_v, r0, r1, o0, o1):
    rows = (r0, r1)
    osem = (o0, o1)

    c = lax.axis_index("c")
    s = lax.axis_index("s")
    w = s * NC + c
    base = w * PER_W

    pltpu.sync_copy(at_hbm, at_v)
    pltpu.sync_copy(nt_hbm, nt_v)
    pltpu.sync_copy(atoms_hbm.at[pl.ds(base, PER_W)], ia_v)
    pltpu.sync_copy(neigh_hbm.at[pl.ds(base, PER_W)], in_v)

    zeros_f = jnp.zeros((16,), jnp.float32)
    for k in range(8):
        at_v[pl.ds(k * 16, 16)] = zeros_f
        nt_v[pl.ds(k * 16, 16)] = zeros_f

    lanes_d = lax.iota(jnp.int32, 16) * D

    def start_out(ci, b):
        dst = out_hbm.at[pl.ds((base + ci * CH) * D, CH * D)]
        pltpu.async_copy(rows[b], dst, osem[b])

    def wait_out(b):
        pltpu.make_async_copy(rows[b], out_hbm.at[pl.ds(0, CH * D)],
                              osem[b]).wait()

    @pl.loop(0, NCHUNK // NBUF)
    def _pair(k):
        for j in range(NBUF):
            ci = k * NBUF + j
            out_v = rows[j]

            @pl.when(k > 0)
            def _():
                wait_out(j)

            tok0 = ci * CH

            @pl.loop(0, CH // 16)
            def _group(g):
                off = tok0 + g * 16
                abase = ia_v[pl.ds(off, 16)] * D
                nbase = in_v[pl.ds(off, 16)] * D
                obase = g * (16 * D) + lanes_d

                @pl.loop(0, D // DQ)
                def _cols(dq):
                    d0 = dq * DQ
                    va = [plsc.load_gather(at_v, [abase + (d0 + jj)])
                          for jj in range(DQ)]
                    vn = [plsc.load_gather(nt_v, [nbase + (d0 + jj)])
                          for jj in range(DQ)]
                    vs = [va[jj] + vn[jj] for jj in range(DQ)]
                    for jj in range(DQ):
                        plsc.store_scatter(out_v, [obase + (d0 + jj)], vs[jj])

            start_out(ci, j)

    for b in range(NBUF):
        wait_out(b)


@jax.jit
def _run(atoms_flat, neigh_flat, at_flat, nt_flat):
    gather = pl.kernel(
        _gather_body,
        out_type=jax.ShapeDtypeStruct((N * D,), jnp.float32),
        mesh=plsc.VectorSubcoreMesh(core_axis_name="c", subcore_axis_name="s"),
        compiler_params=pltpu.CompilerParams(needs_layout_passes=False),
        scratch_types=[
            pltpu.VMEM((AV * D,), jnp.float32),
            pltpu.VMEM((NV * D,), jnp.float32),
            pltpu.VMEM((PER_W,), jnp.int32),
            pltpu.VMEM((PER_W,), jnp.int32),
            pltpu.VMEM((CH * D,), jnp.float32),
            pltpu.VMEM((CH * D,), jnp.float32),
            pltpu.SemaphoreType.DMA,
            pltpu.SemaphoreType.DMA,
        ],
    )
    return gather(atoms_flat, neigh_flat, at_flat, nt_flat)


def kernel(atoms, neighbors, atoms_table, neighbors_table):
    out = _run(atoms.reshape(N), neighbors.reshape(N),
               atoms_table.reshape(AV * D), neighbors_table.reshape(NV * D))
    return out.reshape(B, L, D)


# speedup vs baseline: 10.0577x; 10.0577x over previous
import jax
import jax.numpy as jnp
from jax import lax
from jax.experimental import pallas as pl
from jax.experimental.pallas import tpu as pltpu
from jax.experimental.pallas import tpu_sc as plsc

B, L, D = 4096, 200, 128
N = B * L
NC, NS = 2, 16
NW = NC * NS
PER_W = N // NW
AV, NV = 121, 17
NCOMB = AV * NV
ROWS_W = 65
NCOMB_PAD = ROWS_W * NW
CH = 128
NCHUNK = PER_W // CH
NBUF = 4


def _mesh():
    return plsc.VectorSubcoreMesh(core_axis_name="c", subcore_axis_name="s")


def _wid():
    return lax.axis_index("s") * NC + lax.axis_index("c")


def _build_body(at_hbm, nt_hbm, comb_hbm, at_v, nt_v, buf):
    w = _wid()
    start = w * ROWS_W

    pltpu.sync_copy(at_hbm, at_v)
    pltpu.sync_copy(nt_hbm, nt_v)

    zeros_f = jnp.zeros((16,), jnp.float32)
    for k in range(8):
        at_v[pl.ds(k * 16, 16)] = zeros_f
        nt_v[pl.ds(k * 16, 16)] = zeros_f

    @pl.loop(0, ROWS_W)
    def _row(ri):
        r = start + ri

        @pl.when(r < NCOMB)
        def _():
            a = r // NV
            n = r - a * NV
            for k in range(8):
                va = at_v[pl.ds(a * D + k * 16, 16)]
                vn = nt_v[pl.ds(n * D + k * 16, 16)]
                buf[pl.ds(ri * D + k * 16, 16)] = va + vn

    pltpu.sync_copy(buf, comb_hbm.at[pl.ds(start * D, ROWS_W * D)])


def _gather_body(atoms_hbm, neigh_hbm, comb_hbm, out_hbm,
                 ia_v, in_v, r0, r1, r2, r3, dummy_v, sp_src,
                 g0, g1, g2, g3, o0, o1, o2, o3, dsem):
    rows = (r0, r1, r2, r3)
    gsem = (g0, g1, g2, g3)
    osem = (o0, o1, o2, o3)

    w = _wid()
    base = w * PER_W

    pltpu.sync_copy(atoms_hbm.at[pl.ds(base, PER_W)], ia_v)
    pltpu.sync_copy(neigh_hbm.at[pl.ds(base, PER_W)], in_v)

    @pl.loop(0, PER_W // 16)
    def _fold(i):
        off = i * 16
        ia_v[pl.ds(off, 16)] = ia_v[pl.ds(off, 16)] * NV + in_v[pl.ds(off, 16)]

    def start_gather(ci, b):
        idxs = ia_v.at[pl.ds(ci * CH, CH)]
        pltpu.async_copy(comb_hbm.at[idxs], rows[b], gsem[b])

    def wait_gather(b):
        pltpu.make_async_copy(comb_hbm.at[pl.ds(0, CH)], rows[b],
                              gsem[b]).wait()

    def start_out(ci, b):
        dst = out_hbm.at[pl.ds(base + ci * CH, CH)]
        pltpu.async_copy(rows[b], dst, osem[b])

    def wait_out(b):
        pltpu.make_async_copy(rows[b], out_hbm.at[pl.ds(0, CH)],
                              osem[b]).wait()

    start_gather(0, 0)
    start_gather(1, 1)

    @pl.loop(0, NCHUNK // NBUF)
    def _quad(k):
        for j in range(NBUF):
            ci = k * NBUF + j
            b = j
            b2 = (j + 2) % NBUF
            ci2 = ci + 2

            @pl.when(ci > 0)
            def _():
                pltpu.make_async_copy(sp_src, dummy_v, dsem).wait()

            pltpu.async_copy(sp_src, dummy_v, dsem)

            @pl.when(ci2 >= NBUF)
            def _():
                wait_out(b2)

            @pl.when(ci2 < NCHUNK)
            def _():
                start_gather(ci2, b2)

            wait_gather(b)
            start_out(ci, b)

    pltpu.make_async_copy(sp_src, dummy_v, dsem).wait()
    wait_out((NCHUNK - 2) % NBUF)
    wait_out((NCHUNK - 1) % NBUF)


@jax.jit
def _run(atoms_flat, neigh_flat, at_flat, nt_flat):
    build = pl.kernel(
        _build_body,
        out_type=jax.ShapeDtypeStruct((NCOMB_PAD * D,), jnp.float32),
        mesh=_mesh(),
        compiler_params=pltpu.CompilerParams(needs_layout_passes=False),
        scratch_types=[
            pltpu.VMEM((AV * D,), jnp.float32),
            pltpu.VMEM((NV * D,), jnp.float32),
            pltpu.VMEM((ROWS_W * D,), jnp.float32),
        ],
    )
    comb = build(at_flat, nt_flat).reshape(NCOMB_PAD, D)

    gather = pl.kernel(
        _gather_body,
        out_type=jax.ShapeDtypeStruct((N, D), jnp.float32),
        mesh=_mesh(),
        compiler_params=pltpu.CompilerParams(needs_layout_passes=False),
        scratch_types=[
            pltpu.VMEM((PER_W,), jnp.int32),
            pltpu.VMEM((PER_W,), jnp.int32),
            pltpu.VMEM((CH, D), jnp.float32),
            pltpu.VMEM((CH, D), jnp.float32),
            pltpu.VMEM((CH, D), jnp.float32),
            pltpu.VMEM((CH, D), jnp.float32),
            pltpu.VMEM((8192,), jnp.float32),
            pltpu.VMEM_SHARED((8192,), jnp.float32),
            pltpu.SemaphoreType.DMA,
            pltpu.SemaphoreType.DMA,
            pltpu.SemaphoreType.DMA,
            pltpu.SemaphoreType.DMA,
            pltpu.SemaphoreType.DMA,
            pltpu.SemaphoreType.DMA,
            pltpu.SemaphoreType.DMA,
            pltpu.SemaphoreType.DMA,
            pltpu.SemaphoreType.DMA,
        ],
    )
    return gather(atoms_flat, neigh_flat, comb)


def kernel(atoms, neighbors, atoms_table, neighbors_table):
    out = _run(atoms.reshape(N), neighbors.reshape(N),
               atoms_table.reshape(AV * D), neighbors_table.reshape(NV * D))
    return out.reshape(B, L, D)
